# baseline (speedup 1.0000x reference)
import functools

import jax
import jax.numpy as jnp
from jax import lax
from jax.experimental import pallas as pl
from jax.experimental.pallas import tpu as pltpu
from jax.experimental.pallas import tpu_sc as plsc

N = 10000
IN_CH = 128
OUT_CH = 128
E = 320000

NC = 2
NS = 16
NW = NC * NS

BLK = 64
NROW = 80
NBLK = 2 * NROW
TB = NW * NROW
E_PAD = TB * 128
PAD_N = 10112
ROWS_PER_SUB = PAD_N // NS
HIST_N = 10240
HROWS_PER_SUB = HIST_N // NS

_mesh = plsc.VectorSubcoreMesh(core_axis_name="c", subcore_axis_name="s")


def _zero_2d(ref, rows, cols):
    z = jnp.zeros((16,), jnp.float32)

    def body(r, _):
        for cc in range(cols // 16):
            ref[r, pl.ds(cc * 16, 16)] = z
        return 0

    lax.fori_loop(0, rows, body, 0)


def _unpack_block(packv, j, ib):
    r = j // 2
    h = (j % 2) * BLK
    for k in range(BLK // 16):
        p = packv[r, pl.ds(h + k * 16, 16)]
        ib[0, pl.ds(k * 16, 16)] = jnp.bitwise_and(p, jnp.int32(0xFFFF))
        ib[1, pl.ds(k * 16, 16)] = lax.shift_right_logical(p, jnp.int32(16))


def _stage_blocks(pack_hbm, packv, c, s):
    wid = s * NC + c
    pltpu.sync_copy(pack_hbm.at[wid], packv)


def _deg_body(pack_hbm, out_hbm, packv, dstv, zb, onesv, hist_sh, sem):
    c = lax.axis_index("c")
    s = lax.axis_index("s")

    def zbody(k, _):
        zb[pl.ds(k * 16, 16)] = jnp.zeros((16,), jnp.float32)
        return 0

    lax.fori_loop(0, HROWS_PER_SUB // 16, zbody, 0)
    for k in range(128 // 16):
        onesv[pl.ds(k * 16, 16)] = jnp.ones((16,), jnp.float32)

    pltpu.sync_copy(zb, hist_sh.at[pl.ds(s * HROWS_PER_SUB, HROWS_PER_SUB)])
    _stage_blocks(pack_hbm, packv, c, s)

    def ubody(j, _):
        for k in range(128 // 16):
            p = packv[j, pl.ds(k * 16, 16)]
            dstv[j, pl.ds(k * 16, 16)] = lax.shift_right_logical(
                p, jnp.int32(16)
            )
        return 0

    lax.fori_loop(0, NROW, ubody, 0)
    plsc.subcore_barrier()

    def fbody(j, _):
        pltpu.async_copy(onesv, hist_sh.at[dstv.at[j]], sem, add=True)
        return 0

    lax.fori_loop(0, NROW, fbody, 0)

    def wbody(j, _):
        pltpu.make_async_copy(onesv, hist_sh.at[dstv.at[0]], sem).wait()
        return 0

    lax.fori_loop(0, NROW, wbody, 0)
    plsc.subcore_barrier()
    pltpu.sync_copy(
        hist_sh.at[pl.ds(s * HROWS_PER_SUB, HROWS_PER_SUB)],
        out_hbm.at[c, pl.ds(s * HROWS_PER_SUB, HROWS_PER_SUB)],
    )


_deg_kernel = functools.partial(
    pl.kernel,
    out_type=jax.ShapeDtypeStruct((NC, HIST_N), jnp.float32),
    mesh=_mesh,
    scratch_types=[
        pltpu.VMEM((NROW, 128), jnp.int32),
        pltpu.VMEM((NROW, 128), jnp.int32),
        pltpu.VMEM((HROWS_PER_SUB,), jnp.float32),
        pltpu.VMEM((128,), jnp.float32),
        pltpu.VMEM_SHARED((HIST_N,), jnp.float32),
        pltpu.SemaphoreType.DMA,
    ],
)(_deg_body)


def _agg_body(y_hbm, pack_hbm, out_hbm, packv, ib0, ib1, ib2, ib3,
              rb0, rb1, rb2, rb3, acc_sh, sem0, sem1, sem2, sem3):
    c = lax.axis_index("c")
    s = lax.axis_index("s")

    bufs = (rb0, rb1, rb2, rb3)
    ibs = (ib0, ib1, ib2, ib3)
    sems = (sem0, sem1, sem2, sem3)

    def fire(j, k):
        _unpack_block(packv, j, ibs[k])
        pltpu.async_copy(y_hbm.at[ibs[k].at[0]], bufs[k], sems[k])

    def drain(j, k):
        pltpu.make_async_copy(y_hbm.at[ibs[k].at[0]], bufs[k], sems[k]).wait()
        pltpu.sync_copy(bufs[k], acc_sh.at[ibs[k].at[1]], add=True)

    with jax.named_scope("agg_zero"):
        _stage_blocks(pack_hbm, packv, c, s)
        fire(0, 0)
        fire(1, 1)
        fire(2, 2)
        _zero_2d(rb3, BLK, OUT_CH)
        base = s * ROWS_PER_SUB
        nz = ROWS_PER_SUB // BLK
        for t in range(nz):
            pltpu.async_copy(rb3, acc_sh.at[pl.ds(base + t * BLK, BLK)], sem3)
        rem = ROWS_PER_SUB % BLK
        if rem:
            pltpu.async_copy(
                rb3.at[pl.ds(0, rem)],
                acc_sh.at[pl.ds(base + nz * BLK, rem)],
                sem3,
            )
        for t in range(nz):
            pltpu.make_async_copy(
                rb3, acc_sh.at[pl.ds(base + t * BLK, BLK)], sem3
            ).wait()
        if rem:
            pltpu.make_async_copy(
                rb3.at[pl.ds(0, rem)],
                acc_sh.at[pl.ds(base + nz * BLK, rem)],
                sem3,
            ).wait()
        plsc.subcore_barrier()

    with jax.named_scope("agg_loop"):

        def body(i, _):
            j = 4 * i
            fire(j + 3, 3)
            drain(j, 0)

            @pl.when(j + 4 < NBLK)
            def _():
                fire(j + 4, 0)

            drain(j + 1, 1)

            @pl.when(j + 5 < NBLK)
            def _():
                fire(j + 5, 1)

            drain(j + 2, 2)

            @pl.when(j + 6 < NBLK)
            def _():
                fire(j + 6, 2)

            drain(j + 3, 3)
            return 0

        lax.fori_loop(0, NBLK // 4, body, 0)

    with jax.named_scope("agg_bar"):
        plsc.subcore_barrier()

    with jax.named_scope("agg_out"):
        pltpu.sync_copy(
            acc_sh.at[pl.ds(base, ROWS_PER_SUB)],
            out_hbm.at[c, pl.ds(base, ROWS_PER_SUB)],
        )


_agg_kernel = functools.partial(
    pl.kernel,
    out_type=jax.ShapeDtypeStruct((NC, PAD_N, OUT_CH), jnp.float32),
    mesh=_mesh,
    scratch_types=[
        pltpu.VMEM((NROW, 128), jnp.int32),
        pltpu.VMEM((2, BLK), jnp.int32),
        pltpu.VMEM((2, BLK), jnp.int32),
        pltpu.VMEM((2, BLK), jnp.int32),
        pltpu.VMEM((2, BLK), jnp.int32),
        pltpu.VMEM((BLK, OUT_CH), jnp.float32),
        pltpu.VMEM((BLK, OUT_CH), jnp.float32),
        pltpu.VMEM((BLK, OUT_CH), jnp.float32),
        pltpu.VMEM((BLK, OUT_CH), jnp.float32),
        pltpu.VMEM_SHARED((PAD_N, OUT_CH), jnp.float32),
        pltpu.SemaphoreType.DMA,
        pltpu.SemaphoreType.DMA,
        pltpu.SemaphoreType.DMA,
        pltpu.SemaphoreType.DMA,
    ],
)(_agg_body)


def _pack_body(ei_ref, pk_ref):
    src = ei_ref[0:1, :].reshape(E // 128, 128)
    dst = ei_ref[1:2, :].reshape(E // 128, 128)
    pk_ref[: E // 128, :] = jnp.left_shift(dst, 16) | src
    pr = TB - E // 128
    r2 = jax.lax.broadcasted_iota(jnp.int32, (pr, 128), 0)
    c2 = jax.lax.broadcasted_iota(jnp.int32, (pr, 128), 1)
    k = r2 * 128 + c2
    pad_src = k % N
    pad_dst = N + (k % (PAD_N - N))
    pk_ref[E // 128 :, :] = jnp.left_shift(pad_dst, 16) | pad_src


def _pack_kernel(ei):
    return pl.pallas_call(
        _pack_body,
        out_shape=jax.ShapeDtypeStruct((TB, 128), jnp.int32),
    )(ei)


def _xw_body(x_ref, w_ref, xw_ref):
    xw_ref[...] = jnp.dot(
        x_ref[...], w_ref[...], preferred_element_type=jnp.float32
    )


_RB = 1000


def _xw_kernel(x, W):
    return pl.pallas_call(
        _xw_body,
        grid=(N // _RB,),
        in_specs=[
            pl.BlockSpec((_RB, IN_CH), lambda i: (i, 0)),
            pl.BlockSpec((IN_CH, OUT_CH), lambda i: (0, 0)),
        ],
        out_specs=pl.BlockSpec((_RB, OUT_CH), lambda i: (i, 0)),
        out_shape=jax.ShapeDtypeStruct((N, OUT_CH), jnp.float32),
    )(x, W)


def _scale_body(xw_ref, deg_ref, y_ref, dis_ref):
    deg = deg_ref[0:1, :N] + deg_ref[1:2, :N] + 1.0
    dis = jnp.transpose(lax.rsqrt(deg), (1, 0))
    dis_ref[...] = dis
    y_ref[...] = dis * xw_ref[...]


def _scale_kernel(xw, deg_parts):
    return pl.pallas_call(
        _scale_body,
        out_shape=(
            jax.ShapeDtypeStruct((N, OUT_CH), jnp.float32),
            jax.ShapeDtypeStruct((N, 1), jnp.float32),
        ),
    )(xw, deg_parts)


def _fin_body(a_ref, y_ref, dis_ref, b_ref, o_ref):
    acc = a_ref[0] + a_ref[1] + y_ref[...]
    o_ref[...] = jnp.maximum(dis_ref[...] * acc + b_ref[...], 0.0)


def _fin_kernel(acc_parts, y, dis, b):
    return pl.pallas_call(
        _fin_body,
        grid=(N // _RB,),
        in_specs=[
            pl.BlockSpec((2, _RB, OUT_CH), lambda i: (0, i, 0)),
            pl.BlockSpec((_RB, OUT_CH), lambda i: (i, 0)),
            pl.BlockSpec((_RB, 1), lambda i: (i, 0)),
            pl.BlockSpec((OUT_CH,), lambda i: (0,)),
        ],
        out_specs=pl.BlockSpec((_RB, OUT_CH), lambda i: (i, 0)),
        out_shape=jax.ShapeDtypeStruct((N, OUT_CH), jnp.float32),
    )(acc_parts, y, dis, b)


def kernel(x, edge_index, W, b):
    ei = edge_index.astype(jnp.int32)
    packed = _pack_kernel(ei)
    pack3 = packed.reshape(NW, NROW, 128)
    xw = _xw_kernel(x, W)
    deg_parts = _deg_kernel(pack3)
    y, dis = _scale_kernel(xw, deg_parts)
    acc_parts = _agg_kernel(y, pack3)
    out = _fin_kernel(acc_parts, y, dis, b)
    return out


# speedup vs baseline: 1.0310x; 1.0310x over previous
import functools

import jax
import jax.numpy as jnp
from jax import lax
from jax.experimental import pallas as pl
from jax.experimental.pallas import tpu as pltpu
from jax.experimental.pallas import tpu_sc as plsc

N = 10000
IN_CH = 128
OUT_CH = 128
E = 320000

NC = 2
NS = 16
NW = NC * NS

BLK = 64
NROW = 80
NBLK = 2 * NROW
TB = NW * NROW
E_PAD = TB * 128
PAD_N = 10112
ROWS_PER_SUB = PAD_N // NS
HIST_N = 10240
HROWS_PER_SUB = HIST_N // NS

_mesh = plsc.VectorSubcoreMesh(core_axis_name="c", subcore_axis_name="s")


def _zero_2d(ref, rows, cols):
    z = jnp.zeros((16,), jnp.float32)

    def body(r, _):
        for cc in range(cols // 16):
            ref[r, pl.ds(cc * 16, 16)] = z
        return 0

    lax.fori_loop(0, rows, body, 0)


def _unpack_block(packv, j, ib):
    r = j // 2
    h = (j % 2) * BLK
    for k in range(BLK // 16):
        p = packv[r, pl.ds(h + k * 16, 16)]
        ib[0, pl.ds(k * 16, 16)] = jnp.bitwise_and(p, jnp.int32(0xFFFF))
        ib[1, pl.ds(k * 16, 16)] = lax.shift_right_logical(p, jnp.int32(16))


def _stage_blocks(pack_hbm, packv, c, s):
    wid = s * NC + c
    pltpu.sync_copy(pack_hbm.at[wid], packv)


def _deg_body(pack_hbm, out_hbm, packv, dstv, zb, onesv, hist_sh, sem):
    c = lax.axis_index("c")
    s = lax.axis_index("s")

    def zbody(k, _):
        zb[pl.ds(k * 16, 16)] = jnp.zeros((16,), jnp.float32)
        return 0

    lax.fori_loop(0, HROWS_PER_SUB // 16, zbody, 0)
    for k in range(128 // 16):
        onesv[pl.ds(k * 16, 16)] = jnp.ones((16,), jnp.float32)

    pltpu.sync_copy(zb, hist_sh.at[pl.ds(s * HROWS_PER_SUB, HROWS_PER_SUB)])
    _stage_blocks(pack_hbm, packv, c, s)

    def ubody(j, _):
        for k in range(128 // 16):
            p = packv[j, pl.ds(k * 16, 16)]
            dstv[j, pl.ds(k * 16, 16)] = lax.shift_right_logical(
                p, jnp.int32(16)
            )
        return 0

    lax.fori_loop(0, NROW, ubody, 0)
    plsc.subcore_barrier()

    def fbody(j, _):
        pltpu.async_copy(onesv, hist_sh.at[dstv.at[j]], sem, add=True)
        return 0

    lax.fori_loop(0, NROW, fbody, 0)

    def wbody(j, _):
        pltpu.make_async_copy(onesv, hist_sh.at[dstv.at[0]], sem).wait()
        return 0

    lax.fori_loop(0, NROW, wbody, 0)
    plsc.subcore_barrier()
    pltpu.sync_copy(
        hist_sh.at[pl.ds(s * HROWS_PER_SUB, HROWS_PER_SUB)],
        out_hbm.at[c, pl.ds(s * HROWS_PER_SUB, HROWS_PER_SUB)],
    )


_deg_kernel = functools.partial(
    pl.kernel,
    out_type=jax.ShapeDtypeStruct((NC, HIST_N), jnp.float32),
    mesh=_mesh,
    scratch_types=[
        pltpu.VMEM((NROW, 128), jnp.int32),
        pltpu.VMEM((NROW, 128), jnp.int32),
        pltpu.VMEM((HROWS_PER_SUB,), jnp.float32),
        pltpu.VMEM((128,), jnp.float32),
        pltpu.VMEM_SHARED((HIST_N,), jnp.float32),
        pltpu.SemaphoreType.DMA,
    ],
)(_deg_body)


def _agg_body(y_hbm, pack_hbm, out_hbm, packv, ib0, ib1, ib2, ib3,
              rb0, rb1, rb2, rb3, acc_sh, sem0, sem1, sem2, sem3):
    c = lax.axis_index("c")
    s = lax.axis_index("s")

    bufs = (rb0, rb1, rb2, rb3)
    ibs = (ib0, ib1, ib2, ib3)
    sems = (sem0, sem1, sem2, sem3)

    def fire(j, k):
        _unpack_block(packv, j, ibs[k])
        pltpu.async_copy(y_hbm.at[ibs[k].at[0]], bufs[k], sems[k])

    def drain(j, k):
        pltpu.make_async_copy(y_hbm.at[ibs[k].at[0]], bufs[k], sems[k]).wait()
        pltpu.sync_copy(bufs[k], acc_sh.at[ibs[k].at[1]], add=True)

    with jax.named_scope("agg_zero"):
        _stage_blocks(pack_hbm, packv, c, s)
        fire(0, 0)
        fire(1, 1)
        fire(2, 2)
        _zero_2d(rb3, BLK, OUT_CH)
        base = s * ROWS_PER_SUB
        nz = ROWS_PER_SUB // BLK
        for t in range(nz):
            pltpu.async_copy(rb3, acc_sh.at[pl.ds(base + t * BLK, BLK)], sem3)
        rem = ROWS_PER_SUB % BLK
        if rem:
            pltpu.async_copy(
                rb3.at[pl.ds(0, rem)],
                acc_sh.at[pl.ds(base + nz * BLK, rem)],
                sem3,
            )
        for t in range(nz):
            pltpu.make_async_copy(
                rb3, acc_sh.at[pl.ds(base + t * BLK, BLK)], sem3
            ).wait()
        if rem:
            pltpu.make_async_copy(
                rb3.at[pl.ds(0, rem)],
                acc_sh.at[pl.ds(base + nz * BLK, rem)],
                sem3,
            ).wait()
        plsc.subcore_barrier()

    with jax.named_scope("agg_loop"):

        def body(i, _):
            j = 4 * i
            fire(j + 3, 3)
            drain(j, 0)

            @pl.when(j + 4 < NBLK)
            def _():
                fire(j + 4, 0)

            drain(j + 1, 1)

            @pl.when(j + 5 < NBLK)
            def _():
                fire(j + 5, 1)

            drain(j + 2, 2)

            @pl.when(j + 6 < NBLK)
            def _():
                fire(j + 6, 2)

            drain(j + 3, 3)
            return 0

        lax.fori_loop(0, NBLK // 4, body, 0)

    with jax.named_scope("agg_bar"):
        plsc.subcore_barrier()

    with jax.named_scope("agg_out"):
        pltpu.sync_copy(
            acc_sh.at[pl.ds(base, ROWS_PER_SUB)],
            out_hbm.at[c, pl.ds(base, ROWS_PER_SUB)],
        )


_agg_kernel = functools.partial(
    pl.kernel,
    out_type=jax.ShapeDtypeStruct((NC, PAD_N, OUT_CH), jnp.float32),
    mesh=_mesh,
    scratch_types=[
        pltpu.VMEM((NROW, 128), jnp.int32),
        pltpu.VMEM((2, BLK), jnp.int32),
        pltpu.VMEM((2, BLK), jnp.int32),
        pltpu.VMEM((2, BLK), jnp.int32),
        pltpu.VMEM((2, BLK), jnp.int32),
        pltpu.VMEM((BLK, OUT_CH), jnp.float32),
        pltpu.VMEM((BLK, OUT_CH), jnp.float32),
        pltpu.VMEM((BLK, OUT_CH), jnp.float32),
        pltpu.VMEM((BLK, OUT_CH), jnp.float32),
        pltpu.VMEM_SHARED((PAD_N, OUT_CH), jnp.float32),
        pltpu.SemaphoreType.DMA,
        pltpu.SemaphoreType.DMA,
        pltpu.SemaphoreType.DMA,
        pltpu.SemaphoreType.DMA,
    ],
)(_agg_body)


def _pack_body(ei_ref, pk_ref):
    src = ei_ref[0:1, :].reshape(E // 128, 128)
    dst = ei_ref[1:2, :].reshape(E // 128, 128)
    pk_ref[: E // 128, :] = jnp.left_shift(dst, 16) | src
    pr = TB - E // 128
    r2 = jax.lax.broadcasted_iota(jnp.int32, (pr, 128), 0)
    c2 = jax.lax.broadcasted_iota(jnp.int32, (pr, 128), 1)
    k = r2 * 128 + c2
    pad_src = k % N
    pad_dst = N + (k % (PAD_N - N))
    pk_ref[E // 128 :, :] = jnp.left_shift(pad_dst, 16) | pad_src


def _pack_kernel(ei):
    return pl.pallas_call(
        _pack_body,
        out_shape=jax.ShapeDtypeStruct((TB, 128), jnp.int32),
    )(ei)


def _xw_body(x_ref, w_ref, xw_ref):
    xw_ref[...] = jnp.dot(
        x_ref[...], w_ref[...], preferred_element_type=jnp.float32
    )


def _xw_kernel(x, W):
    return pl.pallas_call(
        _xw_body,
        out_shape=jax.ShapeDtypeStruct((N, OUT_CH), jnp.float32),
    )(x, W)


def _scale_body(xw_ref, deg_ref, y_ref, dis_ref):
    deg = deg_ref[0:1, :N] + deg_ref[1:2, :N] + 1.0
    dis_row = lax.rsqrt(deg)
    dis_ref[...] = dis_row
    y_ref[...] = jnp.transpose(dis_row, (1, 0)) * xw_ref[...]


def _scale_kernel(xw, deg_parts):
    return pl.pallas_call(
        _scale_body,
        out_shape=(
            jax.ShapeDtypeStruct((N, OUT_CH), jnp.float32),
            jax.ShapeDtypeStruct((1, N), jnp.float32),
        ),
    )(xw, deg_parts)


def _fin_body(a_ref, y_ref, dis_ref, b_ref, o_ref):
    acc = a_ref[0, :N, :] + a_ref[1, :N, :] + y_ref[...]
    dis = jnp.transpose(dis_ref[...], (1, 0))
    o_ref[...] = jnp.maximum(dis * acc + b_ref[...], 0.0)


def _fin_kernel(acc_parts, y, dis, b):
    return pl.pallas_call(
        _fin_body,
        out_shape=jax.ShapeDtypeStruct((N, OUT_CH), jnp.float32),
    )(acc_parts, y, dis, b)


def kernel(x, edge_index, W, b):
    ei = edge_index.astype(jnp.int32)
    packed = _pack_kernel(ei)
    pack3 = packed.reshape(NW, NROW, 128)
    xw = _xw_kernel(x, W)
    deg_parts = _deg_kernel(pack3)
    y, dis = _scale_kernel(xw, deg_parts)
    acc_parts = _agg_kernel(y, pack3)
    out = _fin_kernel(acc_parts, y, dis, b)
    return out
